# baseline (speedup 1.0000x reference)
import functools

import jax
import jax.numpy as jnp
from jax import lax
from jax.experimental import pallas as pl
from jax.experimental.pallas import tpu as pltpu
from jax.experimental.pallas import tpu_sc as plsc

_D = 128
_L = 16
_NG = _D // _L
_N_TOK = 32768
_NC = 2
_NS = 16
_NW = _NC * _NS
_TOK_PER_W = _N_TOK // _NW
_CHUNK = 128
_NCHUNK = _TOK_PER_W // _CHUNK
_NBUF = 4


@functools.partial(
    pl.kernel,
    out_type=jax.ShapeDtypeStruct((_N_TOK, _D), jnp.float32),
    mesh=plsc.VectorSubcoreMesh(core_axis_name="c", subcore_axis_name="s"),
    scratch_types=[
        pltpu.VMEM((_TOK_PER_W,), jnp.int32),
        pltpu.VMEM((2, _D), jnp.float32),
        *[pltpu.VMEM((_CHUNK, _D), jnp.float32) for _ in range(_NBUF)],
        *[pltpu.SemaphoreType.DMA for _ in range(_NBUF + 3)],
    ],
)
def _sc_lookup(ids_hbm, table_hbm, out_hbm, ids_v, tab_v,
               buf0, buf1, buf2, buf3,
               sem0, sem1, sem2, sem3, sem_i0, sem_i1, sem_t):
    c = lax.axis_index("c")
    s = lax.axis_index("s")
    wid = s * _NC + c
    base = wid * _TOK_PER_W
    ids_cp0 = pltpu.async_copy(
        ids_hbm.at[pl.ds(base, _CHUNK)], ids_v.at[pl.ds(0, _CHUNK)], sem_i0)
    ids_cp1 = pltpu.async_copy(
        ids_hbm.at[pl.ds(base + _CHUNK, _TOK_PER_W - _CHUNK)],
        ids_v.at[pl.ds(_CHUNK, _TOK_PER_W - _CHUNK)], sem_i1)
    tab_cp = pltpu.async_copy(table_hbm, tab_v, sem_t)
    tab_cp.wait()
    row0 = [tab_v[0, pl.ds(d * _L, _L)] for d in range(_NG)]
    diff = [tab_v[1, pl.ds(d * _L, _L)] - row0[d] for d in range(_NG)]
    ids_cp0.wait()
    bufs, sems = (buf0, buf1, buf2, buf3), (sem0, sem1, sem2, sem3)
    copies = [None] * _NBUF
    for k in range(_NCHUNK):
        b = k % _NBUF
        if copies[b] is not None:
            copies[b].wait()
        if k == 1:
            ids_cp1.wait()
        buf = bufs[b]

        @pl.loop(0, _CHUNK // _L)
        def _grp(g, _k=k, _buf=buf):
            fvec = ids_v[pl.ds(_k * _CHUNK + g * _L, _L)].astype(jnp.float32)
            for j in range(_L):
                f = jnp.zeros((_L,), jnp.float32) + fvec[j]
                for d in range(_NG):
                    _buf[g * _L + j, pl.ds(d * _L, _L)] = row0[d] + f * diff[d]

        copies[b] = pltpu.async_copy(
            buf, out_hbm.at[pl.ds(base + k * _CHUNK, _CHUNK)], sems[b])
    for cp in copies:
        if cp is not None:
            cp.wait()


def kernel(input_ids, embedding_table):
    out = _sc_lookup(input_ids.reshape(-1), embedding_table)
    return out.reshape(input_ids.shape + (_D,)), embedding_table


# speedup vs baseline: 1.1628x; 1.1628x over previous
import functools

import jax
import jax.numpy as jnp
from jax import lax
from jax.experimental import pallas as pl
from jax.experimental.pallas import tpu as pltpu
from jax.experimental.pallas import tpu_sc as plsc

_D = 128
_L = 16
_NG = _D // _L
_N_TOK = 32768
_NC = 2
_NS = 16
_NW = _NC * _NS
_TOK_PER_W = _N_TOK // _NW
_CHUNK = 256
_NCHUNK = _TOK_PER_W // _CHUNK
_NBUF = 3


@functools.partial(
    pl.kernel,
    out_type=jax.ShapeDtypeStruct((_N_TOK, _D), jnp.float32),
    mesh=plsc.VectorSubcoreMesh(core_axis_name="c", subcore_axis_name="s"),
    scratch_types=[
        pltpu.VMEM((_TOK_PER_W,), jnp.int32),
        pltpu.VMEM((2, _D), jnp.float32),
        *[pltpu.VMEM((_CHUNK, _D), jnp.float32) for _ in range(_NBUF)],
        *[pltpu.SemaphoreType.DMA for _ in range(_NBUF + 3)],
    ],
)
def _sc_lookup(ids_hbm, table_hbm, out_hbm, ids_v, tab_v,
               buf0, buf1, buf2,
               sem0, sem1, sem2, sem_i0, sem_i1, sem_t):
    c = lax.axis_index("c")
    s = lax.axis_index("s")
    wid = s * _NC + c
    base = wid * _TOK_PER_W
    ids_cp0 = pltpu.async_copy(
        ids_hbm.at[pl.ds(base, _CHUNK)], ids_v.at[pl.ds(0, _CHUNK)], sem_i0)
    ids_cp1 = pltpu.async_copy(
        ids_hbm.at[pl.ds(base + _CHUNK, _TOK_PER_W - _CHUNK)],
        ids_v.at[pl.ds(_CHUNK, _TOK_PER_W - _CHUNK)], sem_i1)
    tab_cp = pltpu.async_copy(table_hbm, tab_v, sem_t)
    tab_cp.wait()
    row0 = [tab_v[0, pl.ds(d * _L, _L)] for d in range(_NG)]
    diff = [tab_v[1, pl.ds(d * _L, _L)] - row0[d] for d in range(_NG)]
    ids_cp0.wait()
    bufs, sems = (buf0, buf1, buf2), (sem0, sem1, sem2)
    copies = [None] * _NBUF
    for k in range(_NCHUNK):
        b = k % _NBUF
        if copies[b] is not None:
            copies[b].wait()
        if k == 1:
            ids_cp1.wait()
        buf = bufs[b]

        @pl.loop(0, _CHUNK // _L, unroll=2)
        def _grp(g, _k=k, _buf=buf):
            fvec = ids_v[pl.ds(_k * _CHUNK + g * _L, _L)].astype(jnp.float32)
            for j in range(_L):
                f = jnp.zeros((_L,), jnp.float32) + fvec[j]
                for d in range(_NG):
                    _buf[g * _L + j, pl.ds(d * _L, _L)] = row0[d] + f * diff[d]

        copies[b] = pltpu.async_copy(
            buf, out_hbm.at[pl.ds(base + k * _CHUNK, _CHUNK)], sems[b])
    for cp in copies:
        if cp is not None:
            cp.wait()


def kernel(input_ids, embedding_table):
    out = _sc_lookup(input_ids.reshape(-1), embedding_table)
    return out.reshape(input_ids.shape + (_D,)), embedding_table
